# baseline (speedup 1.0000x reference)
import jax
import jax.numpy as jnp
from jax import lax
from jax.experimental import pallas as pl
from jax.experimental.pallas import tpu as pltpu
from jax.experimental.pallas import tpu_sc as plsc

B, T, V, D = 1024, 20, 1000, 64
VP = 1024
TP = 32
NC, NS, L = 2, 16, 16
NW = NC * NS
RW = B // NW
VTAIL = V - (V // L) * L
TTAIL = T - (T // L) * L


def _scores_kernel(ein_ref, eout_ref, ew_ref):
    w = lax.dot_general(
        ein_ref[...], eout_ref[...],
        dimension_numbers=(((1,), (1,)), ((), ())),
        preferred_element_type=jnp.float32)
    ew_ref[...] = jnp.exp(w)


def _gather_kernel(inp_hbm, ew_hbm, voc_hbm, tgt_hbm,
                   dnm_hbm, sxp_hbm,
                   inp_v, erow_v, voc_v, tgt_v, dnm_v, sxp_v, sem):
    wid = lax.axis_index("s") * NC + lax.axis_index("c")
    base = wid * RW

    pltpu.sync_copy(inp_hbm.at[pl.ds(base, RW)], inp_v)
    gat = pltpu.async_copy(ew_hbm.at[inp_v], erow_v, sem)
    pltpu.sync_copy(voc_hbm.at[pl.ds(base, RW)], voc_v)
    pltpu.sync_copy(tgt_hbm.at[pl.ds(base, RW)], tgt_v)
    lane = lax.iota(jnp.int32, L)
    gat.wait()

    def row_body(b, carry):
        rowsplat = jnp.full((L,), b, jnp.int32)

        def chunk(j, acc):
            col = jnp.full((L,), j * L, jnp.int32) + lane
            cidx = plsc.load_gather(voc_v, [rowsplat, col])
            return acc + plsc.load_gather(erow_v, [rowsplat, cidx])

        acc = lax.fori_loop(0, V // L, chunk,
                            jnp.zeros((L,), jnp.float32), unroll=4)
        col = jnp.minimum(jnp.full((L,), (V // L) * L, jnp.int32) + lane,
                          V - 1)
        cidx = plsc.load_gather(voc_v, [rowsplat, col])
        tailv = plsc.load_gather(erow_v, [rowsplat, cidx])
        acc = acc + jnp.where(lane < VTAIL, tailv, 0.0)
        plsc.store_scatter(dnm_v, [rowsplat, lane], acc)

        tidx = plsc.load_gather(tgt_v, [rowsplat, jnp.minimum(lane, T - 1)])
        plsc.store_scatter(sxp_v, [rowsplat, lane],
                           plsc.load_gather(erow_v, [rowsplat, tidx]))
        tidx = plsc.load_gather(tgt_v,
                                [rowsplat, jnp.minimum(lane + L, T - 1)])
        plsc.store_scatter(sxp_v, [rowsplat, lane + L],
                           plsc.load_gather(erow_v, [rowsplat, tidx]))
        return carry

    lax.fori_loop(0, RW, row_body, 0)

    pltpu.sync_copy(dnm_v, dnm_hbm.at[pl.ds(base, RW)])
    pltpu.sync_copy(sxp_v, sxp_hbm.at[pl.ds(base, RW)])


def _loss_kernel(dnm_ref, sxp_ref, out_ref):
    denom = jnp.sum(dnm_ref[...], axis=1, keepdims=True)
    l_denom = jnp.sum(jnp.log(denom))
    col = lax.broadcasted_iota(jnp.int32, (B, TP), 1)
    se = jnp.where(col < T, sxp_ref[...], 1.0)
    l_scores = jnp.sum(jnp.log(se))
    out_ref[...] = jnp.reshape(l_denom / B - l_scores / (B * T), (1, 1))


@jax.jit
def kernel(inputs, targets, vocab, emb_in, emb_out):
    inp = inputs.reshape(B).astype(jnp.int32)
    voc = vocab.astype(jnp.int32)
    tgt = targets.astype(jnp.int32)
    eout = jnp.pad(emb_out, ((0, VP - V), (0, 0)))

    ew = pl.pallas_call(
        _scores_kernel,
        out_shape=jax.ShapeDtypeStruct((V, VP), jnp.float32),
    )(emb_in, eout)

    mesh = plsc.VectorSubcoreMesh(core_axis_name="c", subcore_axis_name="s",
                                  num_cores=NC, num_subcores=NS)
    dnm, sxp = pl.kernel(
        _gather_kernel,
        mesh=mesh,
        compiler_params=pltpu.CompilerParams(use_tc_tiling_on_sc=False,
                                             needs_layout_passes=False),
        out_type=[jax.ShapeDtypeStruct((B, L), jnp.float32),
                  jax.ShapeDtypeStruct((B, TP), jnp.float32)],
        scratch_types=[
            pltpu.VMEM((RW,), jnp.int32),
            pltpu.VMEM((RW, VP), jnp.float32),
            pltpu.VMEM((RW, V), jnp.int32),
            pltpu.VMEM((RW, T), jnp.int32),
            pltpu.VMEM((RW, L), jnp.float32),
            pltpu.VMEM((RW, TP), jnp.float32),
            pltpu.SemaphoreType.DMA,
        ],
    )(inp, ew, voc, tgt)

    loss = pl.pallas_call(
        _loss_kernel,
        out_shape=jax.ShapeDtypeStruct((1, 1), jnp.float32),
    )(dnm, sxp)
    return loss[0, 0]


# speedup vs baseline: 1.1989x; 1.1989x over previous
import jax
import jax.numpy as jnp
from jax import lax
from jax.experimental import pallas as pl
from jax.experimental.pallas import tpu as pltpu
from jax.experimental.pallas import tpu_sc as plsc

B, T, V, D = 1024, 20, 1000, 64
VP = 1024
TP = 32
NC, NS, L = 2, 16, 16
NW = NC * NS
RW = B // NW
VTAIL = V - (V // L) * L
TTAIL = T - (T // L) * L


def _scores_kernel(ein_ref, eout_ref, ew_ref):
    w = lax.dot_general(
        ein_ref[...], eout_ref[...],
        dimension_numbers=(((1,), (1,)), ((), ())),
        preferred_element_type=jnp.float32)
    ew_ref[...] = jnp.exp(w)


def _gather_kernel(inp_hbm, ew_hbm, voc_hbm, tgt_hbm,
                   dnm_hbm, sxp_hbm,
                   inp_v, erow_v, voc_v, tgt_v, dnm_v, sxp_v, sem):
    wid = lax.axis_index("s") * NC + lax.axis_index("c")
    base = wid * RW

    pltpu.sync_copy(inp_hbm.at[pl.ds(base, RW)], inp_v)
    gat = pltpu.async_copy(ew_hbm.at[inp_v], erow_v, sem)
    pltpu.sync_copy(voc_hbm.at[pl.ds(base, RW)], voc_v)
    pltpu.sync_copy(tgt_hbm.at[pl.ds(base, RW)], tgt_v)
    lane = lax.iota(jnp.int32, L)
    gat.wait()

    def row_body(b, carry):
        rowsplat = jnp.full((L,), b, jnp.int32)

        def chunk(j, acc):
            col = jnp.full((L,), j * L, jnp.int32) + lane
            cidx = plsc.load_gather(voc_v, [rowsplat, col])
            return acc + plsc.load_gather(erow_v, [rowsplat, cidx])

        acc = lax.fori_loop(0, V // L, chunk,
                            jnp.zeros((L,), jnp.float32), unroll=4)
        col = jnp.minimum(jnp.full((L,), (V // L) * L, jnp.int32) + lane,
                          V - 1)
        cidx = plsc.load_gather(voc_v, [rowsplat, col])
        tailv = plsc.load_gather(erow_v, [rowsplat, cidx])
        acc = acc + jnp.where(lane < VTAIL, tailv, 0.0)
        plsc.store_scatter(dnm_v, [rowsplat, lane], acc)

        tidx = plsc.load_gather(tgt_v, [rowsplat, jnp.minimum(lane, T - 1)])
        plsc.store_scatter(sxp_v, [rowsplat, lane],
                           plsc.load_gather(erow_v, [rowsplat, tidx]))
        tidx = plsc.load_gather(tgt_v,
                                [rowsplat, jnp.minimum(lane + L, T - 1)])
        plsc.store_scatter(sxp_v, [rowsplat, lane + L],
                           plsc.load_gather(erow_v, [rowsplat, tidx]))
        return carry

    lax.fori_loop(0, RW, row_body, 0)

    pltpu.sync_copy(dnm_v, dnm_hbm.at[pl.ds(base, RW)])
    pltpu.sync_copy(sxp_v, sxp_hbm.at[pl.ds(base, RW)])


def _loss_kernel(dnm_ref, sxp_ref, out_ref):
    denom = jnp.sum(dnm_ref[...], axis=1, keepdims=True)
    l_denom = jnp.sum(jnp.log(denom))
    col = lax.broadcasted_iota(jnp.int32, (B, TP), 1)
    se = jnp.where(col < T, sxp_ref[...], 1.0)
    l_scores = jnp.sum(jnp.log(se))
    out_ref[...] = jnp.reshape(l_denom / B - l_scores / (B * T), (1, 1))


@jax.jit
def kernel(inputs, targets, vocab, emb_in, emb_out):
    inp = inputs.reshape(B).astype(jnp.int32)
    voc = vocab.astype(jnp.int32)
    tgt = targets.astype(jnp.int32)
    eout = jnp.pad(emb_out, ((0, VP - V), (0, 0)))

    ew = pl.pallas_call(
        _scores_kernel,
        out_shape=jax.ShapeDtypeStruct((V, VP), jnp.float32),
    )(emb_in, eout)

    mesh = plsc.VectorSubcoreMesh(core_axis_name="c", subcore_axis_name="s",
                                  num_cores=NC, num_subcores=NS)
    dnm, sxp = pl.kernel(
        _gather_kernel,
        mesh=mesh,
        compiler_params=pltpu.CompilerParams(use_tc_tiling_on_sc=True,
                                             needs_layout_passes=False),
        out_type=[jax.ShapeDtypeStruct((B, L), jnp.float32),
                  jax.ShapeDtypeStruct((B, TP), jnp.float32)],
        scratch_types=[
            pltpu.VMEM((RW,), jnp.int32),
            pltpu.VMEM((RW, VP), jnp.float32),
            pltpu.VMEM((RW, V), jnp.int32),
            pltpu.VMEM((RW, T), jnp.int32),
            pltpu.VMEM((RW, L), jnp.float32),
            pltpu.VMEM((RW, TP), jnp.float32),
            pltpu.SemaphoreType.DMA,
        ],
    )(inp, ew, voc, tgt)

    loss = pl.pallas_call(
        _loss_kernel,
        out_shape=jax.ShapeDtypeStruct((1, 1), jnp.float32),
    )(dnm, sxp)
    return loss[0, 0]
